# Initial kernel scaffold; baseline (speedup 1.0000x reference)
import jax
import jax.numpy as jnp
from jax.experimental import pallas as pl


def kernel(vertices_l0, vertices_l1, rel_pos, norms, w_nl_l0, w_nl_l1, w_rel_l0, w_rel_l1):
    raise NotImplementedError("write your pallas kernel here")


# speedup vs baseline: 16.6906x; 16.6906x over previous
import functools

import jax
import jax.numpy as jnp
from jax import lax
from jax.experimental import pallas as pl

N = 256
C = 4
NB = 8
Y0 = 0.28209479177
Y1 = 0.48860251190
S3 = float(1.0 / jnp.sqrt(3.0))
S2 = float(1.0 / jnp.sqrt(2.0))
_IN = False
SQ3 = 1.7320508075688772
SQ2 = 1.4142135623730951


def _rb(x):
    return x.astype(jnp.bfloat16).astype(jnp.float32)


def _sel(rows, cols):
    r = lax.broadcasted_iota(jnp.int32, (rows, cols), 0)
    c = lax.broadcasted_iota(jnp.int32, (rows, cols), 1)
    return ((c // 16 == r // 4) & (c % 4 == r % 4)).astype(jnp.float32)


def _p1_body(w0_ref, w1_ref, v0_ref, v1_ref):
    r0 = _sel(32, 128)
    r1 = _sel(48, 192).astype(jnp.bfloat16)
    for n in range(NB):
        x0 = w0_ref[n]
        o0 = lax.dot_general(r0, x0, (((1,), (1,)), ((), ())),
                             preferred_element_type=jnp.float32,
                             precision=lax.Precision.HIGHEST)
        v0_ref[:, n, :] = o0
        x1 = w1_ref[n].astype(jnp.bfloat16)
        o1 = lax.dot_general(r1, x1, (((1,), (1,)), ((), ())),
                             preferred_element_type=jnp.float32)
        v1_ref[:, n, :] = o1


def _p1(wr0, wr1):
    return pl.pallas_call(
        _p1_body,
        grid=(N // NB,),
        in_specs=[
            pl.BlockSpec((NB, N, 128), lambda i: (i, 0, 0)),
            pl.BlockSpec((NB, N, 192), lambda i: (i, 0, 0)),
        ],
        out_specs=[
            pl.BlockSpec((32, NB, N), lambda i: (0, i, 0)),
            pl.BlockSpec((48, NB, N), lambda i: (0, i, 0)),
        ],
        out_shape=[
            jax.ShapeDtypeStruct((32, N, N), jnp.float32),
            jax.ShapeDtypeStruct((48, N, N), jnp.float32),
        ],
        interpret=_IN,
    )(wr0, wr1)


def _p2_body(v_ref, nt_ref, wn0_ref, wn1_ref, m_ref):
    conn = (nt_ref[...] < 0.5).astype(jnp.bfloat16)
    vb = v_ref[...].astype(jnp.bfloat16)
    mp0 = lax.dot_general(vb[0:4], conn, (((1,), (0,)), ((), ())),
                          preferred_element_type=jnp.float32)
    mp1 = lax.dot_general(vb[4:16], conn, (((1,), (0,)), ((), ())),
                          preferred_element_type=jnp.float32)
    a0 = mp0
    a1 = mp1.reshape(3, 4, N)
    a1r = _rb(a1)
    p00 = (a0[:, None] * a0[None, :]).reshape(16, N)
    p110 = (jnp.sum(a1r[:, :, None] * a1r[:, None, :], axis=0) / SQ3).reshape(16, N)
    cg0 = jnp.concatenate([p00, p110], axis=0)
    mx0 = jnp.sum(cg0[:, None] * wn0_ref[...], axis=0)
    rows = [mx0]
    wn1 = wn1_ref[...]
    for m in range(3):
        m1, m2 = (m + 1) % 3, (m + 2) % 3
        p01 = (a0[:, None] * a1[m][None, :]).reshape(16, N)
        p10 = (a1[m][:, None] * a0[None, :]).reshape(16, N)
        p111 = ((a1r[m1][:, None] * a1r[m2][None, :]
                 - a1r[m2][:, None] * a1r[m1][None, :]) / SQ2).reshape(16, N)
        cg1 = _rb(jnp.concatenate([p01, p10, p111], axis=0))
        rows.append(jnp.sum(cg1[:, None] * wn1, axis=0))
    m_ref[...] = jnp.concatenate(rows, axis=0)


def _p2(v, normsT, wn0T, wn1T):
    return pl.pallas_call(
        _p2_body,
        out_shape=jax.ShapeDtypeStruct((16, N), jnp.float32),
        interpret=_IN,
    )(v, normsT, wn0T, wn1T)


def _p3_body(m3_ref, v0_ref, v1_ref, u_ref, acc_ref):
    step = pl.program_id(0)

    @pl.when(step == 0)
    def _init():
        acc_ref[...] = jnp.zeros((16, N), jnp.float32)

    mb = m3_ref[0]
    mx0 = mb[0:4]
    mx1 = mb[4:16].reshape(3, 4, NB)
    u = u_ref[...]
    r2 = u[0] * u[0] + u[1] * u[1] + u[2] * u[2]
    r = jnp.sqrt(r2) + 1e-9
    t1 = (u / r[None]) * Y1
    t1r = _rb(t1)
    mx1r = _rb(mx1)
    w0 = v0_ref[...]
    w0a = w0[0:16].reshape(4, 4, NB, N)
    w0b = w0[16:32].reshape(4, 4, NB, N)
    w1 = v1_ref[...]
    w1a = w1[0:16].reshape(4, 4, NB, N)
    w1b = w1[16:32].reshape(4, 4, NB, N)
    w1c = w1[32:48].reshape(4, 4, NB, N)
    f0 = (mx0 * Y0)[:, None, :, None]
    z0 = jnp.sum(w0a * f0, axis=(0, 2))
    a1 = jnp.sum(mx1r[:, :, :, None] * t1r[:, None, :, :], axis=0)
    f0b = a1 / SQ3
    z0b = jnp.sum(w0b * f0b[:, None], axis=(0, 2))
    acc_ref[0:4, :] += z0 + z0b
    rows = []
    for m in range(3):
        m1, m2 = (m + 1) % 3, (m + 2) % 3
        q = _rb(mx0[:, :, None] * t1[m][None])
        t1a = jnp.sum(w1a * q[:, None], axis=(0, 2))
        g = _rb(mx1[m] * Y0)[:, None, :, None]
        t1b = jnp.sum(w1b * g, axis=(0, 2))
        b1m = (mx1r[m1][:, :, None] * t1r[m2][None]
               - mx1r[m2][:, :, None] * t1r[m1][None])
        f1c = _rb(b1m / SQ2)
        t1c = jnp.sum(w1c * f1c[:, None], axis=(0, 2))
        rows.append(t1a + t1b + t1c)
    acc_ref[4:16, :] += jnp.concatenate(rows, axis=0)


def _p3(m3, v0, v1, u):
    return pl.pallas_call(
        _p3_body,
        grid=(N // NB,),
        in_specs=[
            pl.BlockSpec((1, 16, NB), lambda i: (i, 0, 0)),
            pl.BlockSpec((32, NB, N), lambda i: (0, i, 0)),
            pl.BlockSpec((48, NB, N), lambda i: (0, i, 0)),
            pl.BlockSpec((3, NB, N), lambda i: (0, i, 0)),
        ],
        out_specs=pl.BlockSpec((16, N), lambda i: (0, 0)),
        out_shape=jax.ShapeDtypeStruct((16, N), jnp.float32),
        interpret=_IN,
    )(m3, v0, v1, u)


def _p4_body(acc_ref, vn_ref, s_ref):
    acc = acc_ref[...]
    s0 = jnp.sum(acc[0:4])
    s1 = jnp.sum(acc[4:16])
    v0 = acc[0:4] / s0
    v1 = acc[4:16] / s1
    vn_ref[...] = jnp.concatenate([v0, v1], axis=0)
    n0 = (v0[:, None] * v0[None, :]).reshape(16, N)
    v1m = _rb(v1.reshape(3, 4, N))
    n1 = jnp.sum(v1m[:, :, None] * v1m[:, None, :], axis=0).reshape(16, N)
    s_ref[...] = jnp.concatenate([n0, n1], axis=0)


def _p4(acc):
    return pl.pallas_call(
        _p4_body,
        out_shape=[
            jax.ShapeDtypeStruct((16, N), jnp.float32),
            jax.ShapeDtypeStruct((32, N), jnp.float32),
        ],
        interpret=_IN,
    )(acc)


@jax.jit
def kernel(vertices_l0, vertices_l1, rel_pos, norms,
           w_nl_l0, w_nl_l1, w_rel_l0, w_rel_l1):
    normsT = norms.T
    wn0T = w_nl_l0.transpose(1, 2, 0)
    wn1T = _rb(w_nl_l1).transpose(1, 2, 0)
    u = rel_pos[0].transpose(2, 0, 1)[jnp.array([1, 2, 0])]
    v_in = jnp.concatenate(
        [vertices_l0[:, 0, :].T,
         vertices_l1.transpose(1, 2, 0).reshape(12, N)], axis=0)

    w0r, w1r = _p1(w_rel_l0.reshape(N, N, 128), w_rel_l1.reshape(N, N, 192))

    outs = []
    v = v_in
    for _ in range(2):
        m = _p2(v, normsT, wn0T, wn1T)
        m3 = m.reshape(16, N // NB, NB).transpose(1, 0, 2)
        acc = _p3(m3, w0r, w1r, u)
        v, s = _p4(acc)
        part0 = v[0:4].T.reshape(4, N)
        outs.append(jnp.concatenate([part0, s], axis=0))
    return jnp.concatenate(outs, axis=0)
